# baseline (speedup 1.0000x reference)
import functools

import jax
import jax.numpy as jnp
from jax.experimental import pallas as pl
from jax.experimental.pallas import tpu as pltpu

N_CLASS = 100
BIT = 64
NUM_TRAIN = 1000000
BATCH = 16384
M = 0.3

_LOSS_BLK = 2048


def _loss_body(u_ref, y_ref, tv_ref, out_ref):
    y = y_ref[...]
    mx = jnp.max(y, axis=1, keepdims=True)
    ids = jax.lax.broadcasted_iota(jnp.int32, y.shape, 1)
    amax = jnp.min(jnp.where(y >= mx, ids, N_CLASS), axis=1)
    onehot = (ids == amax[:, None]).astype(jnp.float32)
    hc = jax.lax.dot_general(
        onehot, tv_ref[...], (((1,), (0,)), ((), ())),
        preferred_element_type=jnp.float32)
    s = jnp.sum(jnp.maximum(M - u_ref[...] * hc, 0.0))

    @pl.when(pl.program_id(0) == 0)
    def _():
        out_ref[0, 0] = 0.0

    out_ref[0, 0] += s


def _loss(u, y, target_vectors):
    n_blk = BATCH // _LOSS_BLK
    out = pl.pallas_call(
        _loss_body,
        grid=(n_blk,),
        in_specs=[
            pl.BlockSpec((_LOSS_BLK, BIT), lambda i: (i, 0)),
            pl.BlockSpec((_LOSS_BLK, N_CLASS), lambda i: (i, 0)),
            pl.BlockSpec((N_CLASS, BIT), lambda i: (0, 0)),
        ],
        out_specs=pl.BlockSpec(memory_space=pltpu.SMEM),
        out_shape=jax.ShapeDtypeStruct((1, 1), jnp.float32),
    )(u, y, target_vectors)
    return out[0, 0] / (BATCH * BIT)


def kernel(u, y, ind, target_vectors, U, Y):
    loss = _loss(u, y, target_vectors)
    iota = jnp.arange(BATCH, dtype=jnp.int32)
    winner = jnp.full((NUM_TRAIN,), -1, jnp.int32).at[ind].max(iota)
    src = winner[ind]
    U_new = jnp.zeros_like(U).at[ind].set(u[src])
    Y_new = jnp.zeros_like(Y).at[ind].set(y[src])
    return (loss, U_new, Y_new)
